# Initial kernel scaffold; baseline (speedup 1.0000x reference)
import jax
import jax.numpy as jnp
from jax.experimental import pallas as pl


def kernel(coords, input_feat, seps):
    raise NotImplementedError("write your pallas kernel here")


# speedup vs baseline: 3.2309x; 3.2309x over previous
import functools

import jax
import jax.numpy as jnp
from jax import lax
from jax.experimental import pallas as pl
from jax.experimental.pallas import tpu as pltpu
from jax.experimental.pallas import tpu_sc as plsc

_SUB = 2
_L = 16
_NC = 2
_NS = 16
_NW = _NC * _NS


def kernel(coords, input_feat, seps):
    N, D = input_feat.shape
    R = N // _SUB
    B = seps.shape[0]
    W = R * 3
    WIN = N * 3
    assert N % (_SUB * _NW) == 0 and D % _L == 0 and B == _L

    rows_per_tile = R // _NW
    CHUNK = 125
    n_chunks = rows_per_tile // CHUNK
    assert rows_per_tile % CHUNK == 0

    periods = W // 48
    GPT = -(-periods // _NW)
    CWORDS = GPT * 48
    CIN = GPT * 96

    upt = N // _NW
    uvecs = upt // _L

    feat_flat = input_feat.reshape(N * D)
    coords_flat = coords.reshape(WIN)

    mesh = plsc.VectorSubcoreMesh(
        core_axis_name="c", subcore_axis_name="s",
        num_cores=_NC, num_subcores=_NS)

    @functools.partial(
        pl.kernel,
        out_type=[
            jax.ShapeDtypeStruct((R * D,), jnp.float32),
            jax.ShapeDtypeStruct((W,), jnp.float32),
            jax.ShapeDtypeStruct((B,), jnp.int32),
            jax.ShapeDtypeStruct((N,), jnp.int32),
        ],
        mesh=mesh,
        scratch_types=[
            pltpu.VMEM((CHUNK * _SUB * D,), jnp.float32),
            pltpu.VMEM((CHUNK * D,), jnp.float32),
            pltpu.VMEM((CIN,), jnp.float32),
            pltpu.VMEM((CWORDS,), jnp.float32),
            pltpu.VMEM((upt,), jnp.int32),
            pltpu.VMEM((_L,), jnp.int32),
        ],
    )
    def sc_kernel(feat_hbm, coords_hbm, seps_hbm,
                  out_feat, out_coord, out_sep, out_unpool,
                  fin, fout, cin, cout, ubuf, sbuf):
        wid = lax.axis_index("s") * _NC + lax.axis_index("c")
        iota = lax.iota(jnp.int32, _L)

        row0 = wid * rows_per_tile
        D2 = _SUB * D

        @pl.loop(0, n_chunks)
        def _feat(g):
            r0 = row0 + g * CHUNK
            pltpu.sync_copy(feat_hbm.at[pl.ds(r0 * D2, CHUNK * D2)], fin)

            @pl.loop(0, CHUNK)
            def _row(r):
                for q in range(D // _L):
                    a = fin[pl.ds(r * D2 + q * _L, _L)]
                    b = fin[pl.ds(r * D2 + D + q * _L, _L)]
                    fout[pl.ds(r * D + q * _L, _L)] = jnp.maximum(a, b)

            pltpu.sync_copy(fout, out_feat.at[pl.ds(r0 * D, CHUNK * D)])

        p0 = jnp.minimum(wid * GPT, periods - GPT)
        w0 = p0 * 48

        def permute(vec, lane_idx):
            return lax.gather(
                vec, lane_idx[:, None],
                lax.GatherDimensionNumbers(offset_dims=(),
                                           collapsed_slice_dims=(0,),
                                           start_index_map=(0,)),
                (1,), mode=lax.GatherScatterMode.PROMISE_IN_BOUNDS)

        sides = []
        for h in range(3):
            k = h * _L + iota
            d = lax.shift_right_logical(k * 171, 9)
            for off in (0, 3):
                idx = d * 6 + (k - d * 3) + off
                src = lax.shift_right_logical(idx, 4)
                lane = jnp.bitwise_and(idx, _L - 1)
                kk = range(h * _L, (h + 1) * _L)
                srcs = sorted({((j // 3) * 6 + j % 3 + off) // _L for j in kk})
                sides.append((lane, [(j, src == j) for j in srcs]))

        pltpu.sync_copy(coords_hbm.at[pl.ds(w0 * 2, CIN)], cin)

        @pl.loop(0, GPT)
        def _coord(g):
            base = g * 96
            v = [cin[pl.ds(base + j * _L, _L)] for j in range(6)]
            for h in range(3):
                res = []
                for lane, srcs in sides[2 * h:2 * h + 2]:
                    acc = permute(v[srcs[0][0]], lane)
                    for j, m in srcs[1:]:
                        acc = jnp.where(m, permute(v[j], lane), acc)
                    res.append(acc)
                cout[pl.ds(g * 48 + h * _L, _L)] = (res[0] + res[1]) * 0.5

        pltpu.sync_copy(cout, out_coord.at[pl.ds(w0, CWORDS)])

        half = lax.shift_right_logical(iota, 1)
        u0 = wid * (upt // 2)

        @pl.loop(0, uvecs)
        def _unpool(v):
            ubuf[pl.ds(v * _L, _L)] = u0 + v * (_L // 2) + half

        pltpu.sync_copy(ubuf, out_unpool.at[pl.ds(wid * upt, upt)])

        @pl.when(wid == 0)
        def _sep():
            pltpu.sync_copy(seps_hbm, sbuf)
            sbuf[...] = lax.shift_right_logical(sbuf[...] + 1, 1)
            pltpu.sync_copy(sbuf, out_sep)

    rf, rc, rs, ui = sc_kernel(feat_flat, coords_flat, seps)
    return rf.reshape(R, D), rc.reshape(R, 3), rs, ui


# speedup vs baseline: 3.8679x; 1.1971x over previous
import functools

import jax
import jax.numpy as jnp
from jax import lax
from jax.experimental import pallas as pl
from jax.experimental.pallas import tpu as pltpu
from jax.experimental.pallas import tpu_sc as plsc

_SUB = 2
_L = 16
_NC = 2
_NS = 16
_NW = _NC * _NS


def kernel(coords, input_feat, seps):
    N, D = input_feat.shape
    R = N // _SUB
    B = seps.shape[0]
    W = R * 3
    WIN = N * 3
    assert N % (_SUB * _NW) == 0 and D % _L == 0 and B == _L

    rows_per_tile = R // _NW
    CHUNK = 125
    n_chunks = rows_per_tile // CHUNK
    assert rows_per_tile % CHUNK == 0

    periods = W // 48
    GPT = -(-periods // _NW)
    CWORDS = GPT * 48
    CIN = GPT * 96

    upt = N // _NW
    uvecs = upt // _L

    feat_flat = input_feat.reshape(N * D)
    coords_flat = coords.reshape(WIN)

    mesh = plsc.VectorSubcoreMesh(
        core_axis_name="c", subcore_axis_name="s",
        num_cores=_NC, num_subcores=_NS)

    @functools.partial(
        pl.kernel,
        out_type=[
            jax.ShapeDtypeStruct((R * D,), jnp.float32),
            jax.ShapeDtypeStruct((W,), jnp.float32),
            jax.ShapeDtypeStruct((B,), jnp.int32),
            jax.ShapeDtypeStruct((N,), jnp.int32),
        ],
        mesh=mesh,
        scratch_types=[
            pltpu.VMEM((CHUNK * _SUB * D,), jnp.float32),
            pltpu.VMEM((CHUNK * _SUB * D,), jnp.float32),
            pltpu.VMEM((CHUNK * D,), jnp.float32),
            pltpu.VMEM((CHUNK * D,), jnp.float32),
            pltpu.VMEM((upt,), jnp.int32),
            pltpu.VMEM((_L,), jnp.int32),
            pltpu.SemaphoreType.DMA,
            pltpu.SemaphoreType.DMA,
            pltpu.SemaphoreType.DMA,
            pltpu.SemaphoreType.DMA,
        ],
    )
    def sc_kernel(feat_hbm, coords_hbm, seps_hbm,
                  out_feat, out_coord, out_sep, out_unpool,
                  fin0, fin1, fout0, fout1, ubuf, sbuf,
                  si0, si1, so0, so1):
        wid = lax.axis_index("s") * _NC + lax.axis_index("c")
        iota = lax.iota(jnp.int32, _L)

        row0 = wid * rows_per_tile
        D2 = _SUB * D
        n_pairs = n_chunks // 2

        def in_slice(g):
            return feat_hbm.at[pl.ds((row0 + g * CHUNK) * D2, CHUNK * D2)]

        def out_slice(g):
            return out_feat.at[pl.ds((row0 + g * CHUNK) * D, CHUNK * D)]

        def compute(fin, fout):
            @pl.loop(0, CHUNK)
            def _row(r):
                for q in range(D // _L):
                    a = fin[pl.ds(r * D2 + q * _L, _L)]
                    b = fin[pl.ds(r * D2 + D + q * _L, _L)]
                    fout[pl.ds(r * D + q * _L, _L)] = jnp.maximum(a, b)

        pltpu.async_copy(in_slice(0), fin0, si0)
        pltpu.async_copy(in_slice(1), fin1, si1)

        @pl.loop(0, n_pairs)
        def _feat(gg):
            g0 = gg * 2
            for (b, fin, fout, si, so) in ((0, fin0, fout0, si0, so0),
                                           (1, fin1, fout1, si1, so1)):
                @pl.when(gg > 0)
                def _():
                    pltpu.make_async_copy(fout, out_slice(b), so).wait()

                pltpu.make_async_copy(in_slice(b), fin, si).wait()
                compute(fin, fout)
                pltpu.async_copy(fout, out_slice(g0 + b), so)

                @pl.when(gg + 1 < n_pairs)
                def _():
                    pltpu.async_copy(in_slice(g0 + b + 2), fin, si)

        pltpu.make_async_copy(fout0, out_slice(0), so0).wait()
        pltpu.make_async_copy(fout1, out_slice(1), so1).wait()

        p0 = jnp.minimum(wid * GPT, periods - GPT)
        w0 = p0 * 48

        def permute(vec, lane_idx):
            return lax.gather(
                vec, lane_idx[:, None],
                lax.GatherDimensionNumbers(offset_dims=(),
                                           collapsed_slice_dims=(0,),
                                           start_index_map=(0,)),
                (1,), mode=lax.GatherScatterMode.PROMISE_IN_BOUNDS)

        sides = []
        for h in range(3):
            k = h * _L + iota
            d = lax.shift_right_logical(k * 171, 9)
            for off in (0, 3):
                idx = d * 6 + (k - d * 3) + off
                src = lax.shift_right_logical(idx, 4)
                lane = jnp.bitwise_and(idx, _L - 1)
                kk = range(h * _L, (h + 1) * _L)
                srcs = sorted({((j // 3) * 6 + j % 3 + off) // _L for j in kk})
                sides.append((lane, [(j, src == j) for j in srcs]))

        pltpu.sync_copy(coords_hbm.at[pl.ds(w0 * 2, CIN)], fin0.at[pl.ds(0, CIN)])

        @pl.loop(0, GPT)
        def _coord(g):
            base = g * 96
            v = [fin0[pl.ds(base + j * _L, _L)] for j in range(6)]
            for h in range(3):
                res = []
                for lane, srcs in sides[2 * h:2 * h + 2]:
                    acc = permute(v[srcs[0][0]], lane)
                    for j, m in srcs[1:]:
                        acc = jnp.where(m, permute(v[j], lane), acc)
                    res.append(acc)
                fout0[pl.ds(g * 48 + h * _L, _L)] = (res[0] + res[1]) * 0.5

        pltpu.sync_copy(fout0.at[pl.ds(0, CWORDS)], out_coord.at[pl.ds(w0, CWORDS)])

        half = lax.shift_right_logical(iota, 1)
        u0 = wid * (upt // 2)

        @pl.loop(0, uvecs)
        def _unpool(v):
            ubuf[pl.ds(v * _L, _L)] = u0 + v * (_L // 2) + half

        pltpu.sync_copy(ubuf, out_unpool.at[pl.ds(wid * upt, upt)])

        @pl.when(wid == 0)
        def _sep():
            pltpu.sync_copy(seps_hbm, sbuf)
            sbuf[...] = lax.shift_right_logical(sbuf[...] + 1, 1)
            pltpu.sync_copy(sbuf, out_sep)

    rf, rc, rs, ui = sc_kernel(feat_flat, coords_flat, seps)
    return rf.reshape(R, D), rc.reshape(R, 3), rs, ui
